# baseline (speedup 1.0000x reference)
import functools

import jax
import jax.numpy as jnp
from jax import lax
from jax.experimental import pallas as pl
from jax.experimental.pallas import tpu as pltpu
from jax.experimental.pallas import tpu_sc as plsc

N = 10000
E = 160000
L = 128
STEPS = 8

NC = 2
NS = 16
NW = NC * NS
CH = 128
NCH = E // CH
N_PAD = 10240
ROWS_PER_TILE = N_PAD // NS

_sc_mesh = plsc.VectorSubcoreMesh(core_axis_name="c", subcore_axis_name="s")

K = 5
_BIGW = 26
_BIG = 40
_SMALL = 35
IDXW = _BIG * CH


def _worker_span(wid):
    start_c = _BIG * wid - (_BIG - _SMALL) * jnp.maximum(0, wid - _BIGW)
    ngroups = jnp.where(wid < _BIGW, _BIG // K, _SMALL // K)
    return start_c, ngroups


@functools.partial(
    pl.kernel,
    out_type=jax.ShapeDtypeStruct((E, L), jnp.float32),
    mesh=_sc_mesh,
    scratch_types=[
        pltpu.VMEM((K, CH), jnp.int32),
        pltpu.VMEM((K, CH), jnp.int32),
        pltpu.VMEM((K, CH, L), jnp.float32),
        pltpu.SemaphoreType.DMA,
        pltpu.SemaphoreType.DMA,
    ],
)
def _sc_gather_add(a_hbm, b_hbm, src_hbm, dst_hbm, out_hbm, si_v, di_v, buf_v,
                   sem_i, sem):
    wid = lax.axis_index("s") * NC + lax.axis_index("c")
    start_c, ngroups = _worker_span(wid)
    row0 = start_c * CH

    def group(g, _):
        local = g * (K * CH)
        d = []
        for b in range(K):
            r = pl.ds(row0 + local + b * CH, CH)
            d.append(pltpu.async_copy(src_hbm.at[r], si_v.at[b], sem_i))
            d.append(pltpu.async_copy(dst_hbm.at[r], di_v.at[b], sem_i))
        for w in d:
            w.wait()
        d = []
        for b in range(K):
            d.append(pltpu.async_copy(a_hbm.at[si_v.at[b]], buf_v.at[b], sem))
        for w in d:
            w.wait()
        d = []
        for b in range(K):
            d.append(pltpu.async_copy(b_hbm.at[di_v.at[b]], buf_v.at[b], sem, add=True))
        for w in d:
            w.wait()
        d = []
        for b in range(K):
            d.append(pltpu.async_copy(
                buf_v.at[b], out_hbm.at[pl.ds(row0 + local + b * CH, CH)], sem))
        for w in d:
            w.wait()
        return 0

    lax.fori_loop(0, ngroups, group, 0)


KS = 2


@functools.partial(
    pl.kernel,
    out_type=jax.ShapeDtypeStruct((NC, N_PAD, L), jnp.float32),
    mesh=_sc_mesh,
    scratch_types=[
        pltpu.VMEM((KS, CH), jnp.int32),
        pltpu.VMEM((KS, CH, L), jnp.float32),
        pltpu.VMEM_SHARED((N_PAD, L), jnp.float32),
        pltpu.SemaphoreType.DMA,
        pltpu.SemaphoreType.DMA,
    ],
)
def _sc_segment_sum(vals_hbm, dst_hbm, zeros_hbm, out_hbm, idx_v, buf_v, acc_sh,
                    sem_i, sem_v):
    cid = lax.axis_index("c")
    sid = lax.axis_index("s")
    wid = sid * NC + cid
    start_c = 40 * wid - 2 * jnp.maximum(0, wid - 17)
    ngroups = jnp.where(wid < 17, 40 // KS, 38 // KS)
    row0 = start_c * CH

    pltpu.sync_copy(zeros_hbm, buf_v.at[0])
    for b in range(ROWS_PER_TILE // CH):
        pltpu.sync_copy(buf_v.at[0],
                        acc_sh.at[pl.ds(sid * ROWS_PER_TILE + b * CH, CH)])
    plsc.subcore_barrier()

    def group(g, _):
        local = g * (KS * CH)
        d = []
        for b in range(KS):
            r = pl.ds(row0 + local + b * CH, CH)
            d.append(pltpu.async_copy(dst_hbm.at[r], idx_v.at[b], sem_i))
            d.append(pltpu.async_copy(vals_hbm.at[r], buf_v.at[b], sem_v))
        for w in d:
            w.wait()
        for b in range(KS):
            pltpu.sync_copy(buf_v.at[b], acc_sh.at[idx_v.at[b]], add=True)
        return 0

    lax.fori_loop(0, ngroups, group, 0)
    plsc.subcore_barrier()
    pltpu.sync_copy(
        acc_sh.at[pl.ds(sid * ROWS_PER_TILE, ROWS_PER_TILE)],
        out_hbm.at[cid, pl.ds(sid * ROWS_PER_TILE, ROWS_PER_TILE)],
    )


def _leaky(h):
    return jnp.where(h >= 0, h, h * jnp.float32(0.01))


def _ln(h, g, b):
    mu = jnp.mean(h, axis=-1, keepdims=True)
    d = h - mu
    var = jnp.mean(d * d, axis=-1, keepdims=True)
    return d * lax.rsqrt(var + jnp.float32(1e-5)) * g + b


_BN = 2000
_BE = 4000


def _enc_node_body(x_ref, w_ref, b_ref, o_ref):
    o_ref[...] = jnp.dot(x_ref[...], w_ref[...], preferred_element_type=jnp.float32) + b_ref[...]


def _enc_edge_body(a_ref, w_ref, b_ref, o_ref):
    o_ref[...] = jnp.dot(a_ref[...], w_ref[...], preferred_element_type=jnp.float32) + b_ref[...]


def _pre_body(nl_ref, wcat_ref, a_ref, b_ref):
    acc = jnp.dot(nl_ref[...], wcat_ref[...], preferred_element_type=jnp.float32)
    a_ref[...] = acc[:, :L]
    b_ref[...] = acc[:, L:]


def _edge_pre_body(e_ref, w1_ref, b1_ref, c_ref):
    c_ref[...] = (jnp.dot(e_ref[...], w1_ref[...], preferred_element_type=jnp.float32)
                  + b1_ref[...])


def _edge_body(e_ref, c_ref, g_ref, w2_ref, b2_ref, lg_ref, lb_ref, up_ref, new_ref):
    e = e_ref[...]
    h = c_ref[...] + g_ref[...]
    h = _leaky(h)
    h = jnp.dot(h, w2_ref[...], preferred_element_type=jnp.float32) + b2_ref[...]
    h = _leaky(h)
    up = _ln(h, lg_ref[...], lb_ref[...])
    up_ref[...] = up
    new_ref[...] = e + up


def _node_body(p_ref, nl_ref, w1a_ref, w1b_ref, b1_ref, w2_ref, b2_ref, lg_ref, lb_ref, o_ref):
    nl = nl_ref[...]
    agg = p_ref[0] + p_ref[1]
    h = (jnp.dot(nl, w1a_ref[...], preferred_element_type=jnp.float32)
         + jnp.dot(agg, w1b_ref[...], preferred_element_type=jnp.float32)
         + b1_ref[...])
    h = _leaky(h)
    h = jnp.dot(h, w2_ref[...], preferred_element_type=jnp.float32) + b2_ref[...]
    h = _leaky(h)
    o_ref[...] = nl + _ln(h, lg_ref[...], lb_ref[...])


def _dec_body(nl_ref, w1_ref, b1_ref, w2_ref, b2_ref, o_ref):
    h = jnp.dot(nl_ref[...], w1_ref[...], preferred_element_type=jnp.float32) + b1_ref[...]
    h = _leaky(h)
    o_ref[...] = jnp.dot(h, w2_ref[...], preferred_element_type=jnp.float32) + b2_ref[...]


def _full(shape):
    return pl.BlockSpec(shape, lambda j: tuple(0 for _ in shape))


def _rows(block, width):
    return pl.BlockSpec((block, width), lambda j: (j, 0))


_params = pltpu.CompilerParams(dimension_semantics=("parallel",))

_enc_node = pl.pallas_call(
    _enc_node_body,
    grid=(N // _BN,),
    in_specs=[_rows(_BN, L), _full((L, L)), _full((1, L))],
    out_specs=_rows(_BN, L),
    out_shape=jax.ShapeDtypeStruct((N, L), jnp.float32),
    compiler_params=_params,
)

_enc_edge = pl.pallas_call(
    _enc_edge_body,
    grid=(E // _BE,),
    in_specs=[_rows(_BE, 4), _full((4, L)), _full((1, L))],
    out_specs=_rows(_BE, L),
    out_shape=jax.ShapeDtypeStruct((E, L), jnp.float32),
    compiler_params=_params,
)

_pre = pl.pallas_call(
    _pre_body,
    grid=(N // _BN,),
    in_specs=[_rows(_BN, L), _full((L, 2 * L))],
    out_specs=[_rows(_BN, L), _rows(_BN, L)],
    out_shape=[jax.ShapeDtypeStruct((N, L), jnp.float32),
               jax.ShapeDtypeStruct((N, L), jnp.float32)],
    compiler_params=_params,
)

_edge_pre = pl.pallas_call(
    _edge_pre_body,
    grid=(E // _BE,),
    in_specs=[_rows(_BE, L), _full((L, L)), _full((1, L))],
    out_specs=_rows(_BE, L),
    out_shape=jax.ShapeDtypeStruct((E, L), jnp.float32),
    compiler_params=_params,
)

_edge_mlp = pl.pallas_call(
    _edge_body,
    grid=(E // _BE,),
    in_specs=[_rows(_BE, L), _rows(_BE, L), _rows(_BE, L),
              _full((L, L)), _full((1, L)), _full((1, L)), _full((1, L))],
    out_specs=[_rows(_BE, L), _rows(_BE, L)],
    out_shape=[jax.ShapeDtypeStruct((E, L), jnp.float32),
               jax.ShapeDtypeStruct((E, L), jnp.float32)],
    compiler_params=_params,
)

_node_mlp = pl.pallas_call(
    _node_body,
    grid=(N // _BN,),
    in_specs=[pl.BlockSpec((NC, _BN, L), lambda j: (0, j, 0)),
              _rows(_BN, L), _full((L, L)), _full((L, L)), _full((1, L)),
              _full((L, L)), _full((1, L)), _full((1, L)), _full((1, L))],
    out_specs=_rows(_BN, L),
    out_shape=jax.ShapeDtypeStruct((N, L), jnp.float32),
    compiler_params=_params,
)

_dec = pl.pallas_call(
    _dec_body,
    grid=(N // _BN,),
    in_specs=[_rows(_BN, L), _full((L, L)), _full((1, L)), _full((L, 3)), _full((1, 3))],
    out_specs=_rows(_BN, 3),
    out_shape=jax.ShapeDtypeStruct((N, 3), jnp.float32),
    compiler_params=_params,
)


def kernel(x, edge_index, edge_attr, enc_node_W, enc_node_b, enc_edge_W, enc_edge_b,
           edge_W1, edge_b1, edge_W2, edge_b2, edge_ln_g, edge_ln_b,
           node_W1, node_b1, node_W2, node_b2, node_ln_g, node_ln_b,
           dec_W1, dec_b1, dec_W2, dec_b2):
    src = edge_index[0].astype(jnp.int32)
    dst = edge_index[1].astype(jnp.int32)

    node_lat = _enc_node(x, enc_node_W, enc_node_b.reshape(1, L))
    edge_lat = _enc_edge(edge_attr, enc_edge_W, enc_edge_b.reshape(1, L))

    zeros_tile = jnp.zeros((CH, L), jnp.float32)

    w1a = edge_W1[:, :L, :]
    wbc = jnp.concatenate([edge_W1[:, L:2 * L, :], edge_W1[:, 2 * L:, :]], axis=2)
    xs = (w1a, wbc, edge_b1.reshape(STEPS, 1, L), edge_W2,
          edge_b2.reshape(STEPS, 1, L), edge_ln_g.reshape(STEPS, 1, L),
          edge_ln_b.reshape(STEPS, 1, L),
          node_W1[:, :L, :], node_W1[:, L:, :], node_b1.reshape(STEPS, 1, L),
          node_W2, node_b2.reshape(STEPS, 1, L), node_ln_g.reshape(STEPS, 1, L),
          node_ln_b.reshape(STEPS, 1, L))

    def step(carry, ws):
        nl, el = carry
        (sw1a, swbc, sb1, sw2, sb2, slg, slb,
         snw1a, snw1b, snb1, snw2, snb2, snlg, snlb) = ws
        a_t, b_t = _pre(nl, swbc)
        gath = _sc_gather_add(a_t, b_t, src, dst)
        c_t = _edge_pre(el, sw1a, sb1)
        e_upd, el_new = _edge_mlp(el, c_t, gath, sw2, sb2, slg, slb)
        parts = _sc_segment_sum(e_upd, dst, zeros_tile)
        nl_new = _node_mlp(parts, nl, snw1a, snw1b, snb1, snw2, snb2, snlg, snlb)
        return (nl_new, el_new), None

    (node_lat, _), _ = lax.scan(step, (node_lat, edge_lat), xs)

    return _dec(node_lat, dec_W1, dec_b1.reshape(1, L), dec_W2, dec_b2.reshape(1, 3))


# speedup vs baseline: 1.1667x; 1.1667x over previous
import functools

import jax
import jax.numpy as jnp
from jax import lax
from jax.experimental import pallas as pl
from jax.experimental.pallas import tpu as pltpu
from jax.experimental.pallas import tpu_sc as plsc

N = 10000
E = 160000
L = 128
STEPS = 8

NC = 2
NS = 16
NW = NC * NS
CH = 128
NCH = E // CH
N_PAD = 10240
ROWS_PER_TILE = N_PAD // NS

_sc_mesh = plsc.VectorSubcoreMesh(core_axis_name="c", subcore_axis_name="s")

K = 5
_BIGW = 26
_BIG = 40
_SMALL = 35
IDXW = _BIG * CH


def _worker_span(wid):
    start_c = _BIG * wid - (_BIG - _SMALL) * jnp.maximum(0, wid - _BIGW)
    ngroups = jnp.where(wid < _BIGW, _BIG // K, _SMALL // K)
    return start_c, ngroups


@functools.partial(
    pl.kernel,
    out_type=jax.ShapeDtypeStruct((E, L), jnp.float32),
    mesh=_sc_mesh,
    scratch_types=[
        pltpu.VMEM((K, CH), jnp.int32),
        pltpu.VMEM((K, CH), jnp.int32),
        pltpu.VMEM((K, CH, L), jnp.float32),
        pltpu.SemaphoreType.DMA,
        pltpu.SemaphoreType.DMA,
    ],
)
def _sc_gather_add(a_hbm, b_hbm, src_hbm, dst_hbm, out_hbm, si_v, di_v, buf_v,
                   sem_i, sem):
    wid = lax.axis_index("s") * NC + lax.axis_index("c")
    start_c, ngroups = _worker_span(wid)
    row0 = start_c * CH

    def group(g, _):
        local = g * (K * CH)
        d = []
        for b in range(K):
            r = pl.ds(row0 + local + b * CH, CH)
            d.append(pltpu.async_copy(src_hbm.at[r], si_v.at[b], sem_i))
            d.append(pltpu.async_copy(dst_hbm.at[r], di_v.at[b], sem_i))
        for w in d:
            w.wait()
        d = []
        for b in range(K):
            d.append(pltpu.async_copy(a_hbm.at[si_v.at[b]], buf_v.at[b], sem))
        for w in d:
            w.wait()
        d = []
        for b in range(K):
            d.append(pltpu.async_copy(b_hbm.at[di_v.at[b]], buf_v.at[b], sem, add=True))
        for w in d:
            w.wait()
        d = []
        for b in range(K):
            d.append(pltpu.async_copy(
                buf_v.at[b], out_hbm.at[pl.ds(row0 + local + b * CH, CH)], sem))
        for w in d:
            w.wait()
        return 0

    lax.fori_loop(0, ngroups, group, 0)


KS = 2


@functools.partial(
    pl.kernel,
    out_type=jax.ShapeDtypeStruct((NC, N_PAD, L), jnp.float32),
    mesh=_sc_mesh,
    scratch_types=[
        pltpu.VMEM((KS, CH), jnp.int32),
        pltpu.VMEM((KS, CH, L), jnp.float32),
        pltpu.VMEM_SHARED((N_PAD, L), jnp.float32),
        pltpu.SemaphoreType.DMA,
        pltpu.SemaphoreType.DMA,
    ],
)
def _sc_segment_sum(vals_hbm, dst_hbm, zeros_hbm, out_hbm, idx_v, buf_v, acc_sh,
                    sem_i, sem_v):
    cid = lax.axis_index("c")
    sid = lax.axis_index("s")
    wid = sid * NC + cid
    start_c = 40 * wid - 2 * jnp.maximum(0, wid - 17)
    ngroups = jnp.where(wid < 17, 40 // KS, 38 // KS)
    row0 = start_c * CH

    pltpu.sync_copy(zeros_hbm, buf_v.at[0])
    for b in range(ROWS_PER_TILE // CH):
        pltpu.sync_copy(buf_v.at[0],
                        acc_sh.at[pl.ds(sid * ROWS_PER_TILE + b * CH, CH)])
    plsc.subcore_barrier()

    def group(g, _):
        local = g * (KS * CH)
        d = []
        for b in range(KS):
            r = pl.ds(row0 + local + b * CH, CH)
            d.append(pltpu.async_copy(dst_hbm.at[r], idx_v.at[b], sem_i))
            d.append(pltpu.async_copy(vals_hbm.at[r], buf_v.at[b], sem_v))
        for w in d:
            w.wait()
        d = []
        for b in range(KS):
            d.append(pltpu.async_copy(buf_v.at[b], acc_sh.at[idx_v.at[b]],
                                      sem_v, add=True))
        for w in d:
            w.wait()
        return 0

    lax.fori_loop(0, ngroups, group, 0)
    plsc.subcore_barrier()
    pltpu.sync_copy(
        acc_sh.at[pl.ds(sid * ROWS_PER_TILE, ROWS_PER_TILE)],
        out_hbm.at[cid, pl.ds(sid * ROWS_PER_TILE, ROWS_PER_TILE)],
    )


def _leaky(h):
    return jnp.where(h >= 0, h, h * jnp.float32(0.01))


def _ln(h, g, b):
    mu = jnp.mean(h, axis=-1, keepdims=True)
    d = h - mu
    var = jnp.mean(d * d, axis=-1, keepdims=True)
    return d * lax.rsqrt(var + jnp.float32(1e-5)) * g + b


_BN = 2000
_BE = 4000


def _enc_node_body(x_ref, w_ref, b_ref, o_ref):
    o_ref[...] = jnp.dot(x_ref[...], w_ref[...], preferred_element_type=jnp.float32) + b_ref[...]


def _enc_edge_body(a_ref, w_ref, b_ref, o_ref):
    o_ref[...] = jnp.dot(a_ref[...], w_ref[...], preferred_element_type=jnp.float32) + b_ref[...]


def _pre_body(nl_ref, wcat_ref, a_ref, b_ref):
    acc = jnp.dot(nl_ref[...], wcat_ref[...], preferred_element_type=jnp.float32)
    a_ref[...] = acc[:, :L]
    b_ref[...] = acc[:, L:]


def _edge_body(e_ref, g_ref, w1_ref, b1_ref, w2_ref, b2_ref, lg_ref, lb_ref, up_ref, new_ref):
    e = e_ref[...]
    h = jnp.dot(e, w1_ref[...], preferred_element_type=jnp.float32) + g_ref[...] + b1_ref[...]
    h = _leaky(h)
    h = jnp.dot(h, w2_ref[...], preferred_element_type=jnp.float32) + b2_ref[...]
    h = _leaky(h)
    up = _ln(h, lg_ref[...], lb_ref[...])
    up_ref[...] = up
    new_ref[...] = e + up


def _node_body(p_ref, nl_ref, w1a_ref, w1b_ref, b1_ref, w2_ref, b2_ref, lg_ref, lb_ref, o_ref):
    nl = nl_ref[...]
    agg = p_ref[0] + p_ref[1]
    h = (jnp.dot(nl, w1a_ref[...], preferred_element_type=jnp.float32)
         + jnp.dot(agg, w1b_ref[...], preferred_element_type=jnp.float32)
         + b1_ref[...])
    h = _leaky(h)
    h = jnp.dot(h, w2_ref[...], preferred_element_type=jnp.float32) + b2_ref[...]
    h = _leaky(h)
    o_ref[...] = nl + _ln(h, lg_ref[...], lb_ref[...])


def _dec_body(nl_ref, w1_ref, b1_ref, w2_ref, b2_ref, o_ref):
    h = jnp.dot(nl_ref[...], w1_ref[...], preferred_element_type=jnp.float32) + b1_ref[...]
    h = _leaky(h)
    o_ref[...] = jnp.dot(h, w2_ref[...], preferred_element_type=jnp.float32) + b2_ref[...]


def _full(shape):
    return pl.BlockSpec(shape, lambda j: tuple(0 for _ in shape))


def _rows(block, width):
    return pl.BlockSpec((block, width), lambda j: (j, 0))


_params = pltpu.CompilerParams(dimension_semantics=("parallel",))

_enc_node = pl.pallas_call(
    _enc_node_body,
    grid=(N // _BN,),
    in_specs=[_rows(_BN, L), _full((L, L)), _full((1, L))],
    out_specs=_rows(_BN, L),
    out_shape=jax.ShapeDtypeStruct((N, L), jnp.float32),
    compiler_params=_params,
)

_enc_edge = pl.pallas_call(
    _enc_edge_body,
    grid=(E // _BE,),
    in_specs=[_rows(_BE, 4), _full((4, L)), _full((1, L))],
    out_specs=_rows(_BE, L),
    out_shape=jax.ShapeDtypeStruct((E, L), jnp.float32),
    compiler_params=_params,
)

_pre = pl.pallas_call(
    _pre_body,
    grid=(N // _BN,),
    in_specs=[_rows(_BN, L), _full((L, 2 * L))],
    out_specs=[_rows(_BN, L), _rows(_BN, L)],
    out_shape=[jax.ShapeDtypeStruct((N, L), jnp.float32),
               jax.ShapeDtypeStruct((N, L), jnp.float32)],
    compiler_params=_params,
)

_edge_mlp = pl.pallas_call(
    _edge_body,
    grid=(E // _BE,),
    in_specs=[_rows(_BE, L), _rows(_BE, L), _full((L, L)), _full((1, L)),
              _full((L, L)), _full((1, L)), _full((1, L)), _full((1, L))],
    out_specs=[_rows(_BE, L), _rows(_BE, L)],
    out_shape=[jax.ShapeDtypeStruct((E, L), jnp.float32),
               jax.ShapeDtypeStruct((E, L), jnp.float32)],
    compiler_params=_params,
)

_node_mlp = pl.pallas_call(
    _node_body,
    grid=(N // _BN,),
    in_specs=[pl.BlockSpec((NC, _BN, L), lambda j: (0, j, 0)),
              _rows(_BN, L), _full((L, L)), _full((L, L)), _full((1, L)),
              _full((L, L)), _full((1, L)), _full((1, L)), _full((1, L))],
    out_specs=_rows(_BN, L),
    out_shape=jax.ShapeDtypeStruct((N, L), jnp.float32),
    compiler_params=_params,
)

_dec = pl.pallas_call(
    _dec_body,
    grid=(N // _BN,),
    in_specs=[_rows(_BN, L), _full((L, L)), _full((1, L)), _full((L, 3)), _full((1, 3))],
    out_specs=_rows(_BN, 3),
    out_shape=jax.ShapeDtypeStruct((N, 3), jnp.float32),
    compiler_params=_params,
)


def kernel(x, edge_index, edge_attr, enc_node_W, enc_node_b, enc_edge_W, enc_edge_b,
           edge_W1, edge_b1, edge_W2, edge_b2, edge_ln_g, edge_ln_b,
           node_W1, node_b1, node_W2, node_b2, node_ln_g, node_ln_b,
           dec_W1, dec_b1, dec_W2, dec_b2):
    src = edge_index[0].astype(jnp.int32)
    dst = edge_index[1].astype(jnp.int32)

    node_lat = _enc_node(x, enc_node_W, enc_node_b.reshape(1, L))
    edge_lat = _enc_edge(edge_attr, enc_edge_W, enc_edge_b.reshape(1, L))

    zeros_tile = jnp.zeros((CH, L), jnp.float32)

    w1a = edge_W1[:, :L, :]
    wbc = jnp.concatenate([edge_W1[:, L:2 * L, :], edge_W1[:, 2 * L:, :]], axis=2)
    xs = (w1a, wbc, edge_b1.reshape(STEPS, 1, L), edge_W2,
          edge_b2.reshape(STEPS, 1, L), edge_ln_g.reshape(STEPS, 1, L),
          edge_ln_b.reshape(STEPS, 1, L),
          node_W1[:, :L, :], node_W1[:, L:, :], node_b1.reshape(STEPS, 1, L),
          node_W2, node_b2.reshape(STEPS, 1, L), node_ln_g.reshape(STEPS, 1, L),
          node_ln_b.reshape(STEPS, 1, L))

    def step(carry, ws):
        nl, el = carry
        (sw1a, swbc, sb1, sw2, sb2, slg, slb,
         snw1a, snw1b, snb1, snw2, snb2, snlg, snlb) = ws
        a_t, b_t = _pre(nl, swbc)
        gath = _sc_gather_add(a_t, b_t, src, dst)
        e_upd, el_new = _edge_mlp(el, gath, sw1a, sb1, sw2, sb2, slg, slb)
        parts = _sc_segment_sum(e_upd, dst, zeros_tile)
        nl_new = _node_mlp(parts, nl, snw1a, snw1b, snb1, snw2, snb2, snlg, snlb)
        return (nl_new, el_new), None

    (node_lat, _), _ = lax.scan(step, (node_lat, edge_lat), xs)

    return _dec(node_lat, dec_W1, dec_b1.reshape(1, L), dec_W2, dec_b2.reshape(1, 3))


# speedup vs baseline: 1.1800x; 1.0114x over previous
import functools

import jax
import jax.numpy as jnp
from jax import lax
from jax.experimental import pallas as pl
from jax.experimental.pallas import tpu as pltpu
from jax.experimental.pallas import tpu_sc as plsc

N = 10000
E = 160000
L = 128
STEPS = 8

NC = 2
NS = 16
NW = NC * NS
CH = 128
NCH = E // CH
N_PAD = 10240
ROWS_PER_TILE = N_PAD // NS

_sc_mesh = plsc.VectorSubcoreMesh(core_axis_name="c", subcore_axis_name="s")

K = 5
_BIGW = 26
_BIG = 40
_SMALL = 35
IDXW = _BIG * CH


def _worker_span(wid):
    start_c = _BIG * wid - (_BIG - _SMALL) * jnp.maximum(0, wid - _BIGW)
    ngroups = jnp.where(wid < _BIGW, _BIG // K, _SMALL // K)
    return start_c, ngroups


@functools.partial(
    pl.kernel,
    out_type=jax.ShapeDtypeStruct((E, L), jnp.float32),
    mesh=_sc_mesh,
    scratch_types=[
        pltpu.VMEM((K, CH), jnp.int32),
        pltpu.VMEM((K, CH), jnp.int32),
        pltpu.VMEM((K, CH, L), jnp.float32),
        pltpu.SemaphoreType.DMA,
        pltpu.SemaphoreType.DMA,
        pltpu.SemaphoreType.DMA,
    ],
)
def _sc_gather_add(a_hbm, b_hbm, src_hbm, dst_hbm, out_hbm, si_v, di_v, buf_v,
                   sem_i, sem, sem_w):
    wid = lax.axis_index("s") * NC + lax.axis_index("c")
    start_c, ngroups = _worker_span(wid)
    row0 = start_c * CH

    def group(g, _):
        local = g * (K * CH)
        d = []
        for b in range(K):
            r = pl.ds(row0 + local + b * CH, CH)
            d.append(pltpu.async_copy(src_hbm.at[r], si_v.at[b], sem_i))
            d.append(pltpu.async_copy(dst_hbm.at[r], di_v.at[b], sem_i))

        @pl.when(g > 0)
        def _drain_wb():
            for b in range(K):
                pltpu.make_async_copy(
                    buf_v.at[b], out_hbm.at[pl.ds(row0, CH)], sem_w).wait()

        for w in d:
            w.wait()
        d = []
        for b in range(K):
            d.append(pltpu.async_copy(a_hbm.at[si_v.at[b]], buf_v.at[b], sem))
        for w in d:
            w.wait()
        d = []
        for b in range(K):
            d.append(pltpu.async_copy(b_hbm.at[di_v.at[b]], buf_v.at[b], sem, add=True))
        for w in d:
            w.wait()
        for b in range(K):
            pltpu.async_copy(
                buf_v.at[b], out_hbm.at[pl.ds(row0 + local + b * CH, CH)], sem_w)
        return 0

    lax.fori_loop(0, ngroups, group, 0)
    for b in range(K):
        pltpu.make_async_copy(buf_v.at[b], out_hbm.at[pl.ds(row0, CH)], sem_w).wait()


KS = 2


@functools.partial(
    pl.kernel,
    out_type=jax.ShapeDtypeStruct((NC, N_PAD, L), jnp.float32),
    mesh=_sc_mesh,
    scratch_types=[
        pltpu.VMEM((KS, CH), jnp.int32),
        pltpu.VMEM((KS, CH, L), jnp.float32),
        pltpu.VMEM_SHARED((N_PAD, L), jnp.float32),
        pltpu.SemaphoreType.DMA,
        pltpu.SemaphoreType.DMA,
    ],
)
def _sc_segment_sum(vals_hbm, dst_hbm, zeros_hbm, out_hbm, idx_v, buf_v, acc_sh,
                    sem_i, sem_v):
    cid = lax.axis_index("c")
    sid = lax.axis_index("s")
    wid = sid * NC + cid
    start_c = 40 * wid - 2 * jnp.maximum(0, wid - 17)
    ngroups = jnp.where(wid < 17, 40 // KS, 38 // KS)
    row0 = start_c * CH

    pltpu.sync_copy(zeros_hbm, buf_v.at[0])
    for b in range(ROWS_PER_TILE // CH):
        pltpu.sync_copy(buf_v.at[0],
                        acc_sh.at[pl.ds(sid * ROWS_PER_TILE + b * CH, CH)])
    plsc.subcore_barrier()

    def group(g, _):
        local = g * (KS * CH)
        d = []
        for b in range(KS):
            r = pl.ds(row0 + local + b * CH, CH)
            d.append(pltpu.async_copy(dst_hbm.at[r], idx_v.at[b], sem_i))
            d.append(pltpu.async_copy(vals_hbm.at[r], buf_v.at[b], sem_v))
        for w in d:
            w.wait()
        d = []
        for b in range(KS):
            d.append(pltpu.async_copy(buf_v.at[b], acc_sh.at[idx_v.at[b]],
                                      sem_v, add=True))
        for w in d:
            w.wait()
        return 0

    lax.fori_loop(0, ngroups, group, 0)
    plsc.subcore_barrier()
    pltpu.sync_copy(
        acc_sh.at[pl.ds(sid * ROWS_PER_TILE, ROWS_PER_TILE)],
        out_hbm.at[cid, pl.ds(sid * ROWS_PER_TILE, ROWS_PER_TILE)],
    )


def _leaky(h):
    return jnp.where(h >= 0, h, h * jnp.float32(0.01))


def _ln(h, g, b):
    mu = jnp.mean(h, axis=-1, keepdims=True)
    d = h - mu
    var = jnp.mean(d * d, axis=-1, keepdims=True)
    return d * lax.rsqrt(var + jnp.float32(1e-5)) * g + b


_BN = 2000
_BE = 4000


def _enc_node_body(x_ref, w_ref, b_ref, o_ref):
    o_ref[...] = jnp.dot(x_ref[...], w_ref[...], preferred_element_type=jnp.float32) + b_ref[...]


def _enc_edge_body(a_ref, w_ref, b_ref, o_ref):
    o_ref[...] = jnp.dot(a_ref[...], w_ref[...], preferred_element_type=jnp.float32) + b_ref[...]


def _pre_body(nl_ref, wcat_ref, a_ref, b_ref):
    acc = jnp.dot(nl_ref[...], wcat_ref[...], preferred_element_type=jnp.float32)
    a_ref[...] = acc[:, :L]
    b_ref[...] = acc[:, L:]


def _edge_body(e_ref, g_ref, w1_ref, b1_ref, w2_ref, b2_ref, lg_ref, lb_ref, up_ref, new_ref):
    e = e_ref[...]
    h = jnp.dot(e, w1_ref[...], preferred_element_type=jnp.float32) + g_ref[...] + b1_ref[...]
    h = _leaky(h)
    h = jnp.dot(h, w2_ref[...], preferred_element_type=jnp.float32) + b2_ref[...]
    h = _leaky(h)
    up = _ln(h, lg_ref[...], lb_ref[...])
    up_ref[...] = up
    new_ref[...] = e + up


def _node_body(p_ref, nl_ref, w1a_ref, w1b_ref, b1_ref, w2_ref, b2_ref, lg_ref, lb_ref, o_ref):
    nl = nl_ref[...]
    agg = p_ref[0] + p_ref[1]
    h = (jnp.dot(nl, w1a_ref[...], preferred_element_type=jnp.float32)
         + jnp.dot(agg, w1b_ref[...], preferred_element_type=jnp.float32)
         + b1_ref[...])
    h = _leaky(h)
    h = jnp.dot(h, w2_ref[...], preferred_element_type=jnp.float32) + b2_ref[...]
    h = _leaky(h)
    o_ref[...] = nl + _ln(h, lg_ref[...], lb_ref[...])


def _dec_body(nl_ref, w1_ref, b1_ref, w2_ref, b2_ref, o_ref):
    h = jnp.dot(nl_ref[...], w1_ref[...], preferred_element_type=jnp.float32) + b1_ref[...]
    h = _leaky(h)
    o_ref[...] = jnp.dot(h, w2_ref[...], preferred_element_type=jnp.float32) + b2_ref[...]


def _full(shape):
    return pl.BlockSpec(shape, lambda j: tuple(0 for _ in shape))


def _rows(block, width):
    return pl.BlockSpec((block, width), lambda j: (j, 0))


_params = pltpu.CompilerParams(dimension_semantics=("parallel",))

_enc_node = pl.pallas_call(
    _enc_node_body,
    grid=(N // _BN,),
    in_specs=[_rows(_BN, L), _full((L, L)), _full((1, L))],
    out_specs=_rows(_BN, L),
    out_shape=jax.ShapeDtypeStruct((N, L), jnp.float32),
    compiler_params=_params,
)

_enc_edge = pl.pallas_call(
    _enc_edge_body,
    grid=(E // _BE,),
    in_specs=[_rows(_BE, 4), _full((4, L)), _full((1, L))],
    out_specs=_rows(_BE, L),
    out_shape=jax.ShapeDtypeStruct((E, L), jnp.float32),
    compiler_params=_params,
)

_pre = pl.pallas_call(
    _pre_body,
    grid=(N // _BN,),
    in_specs=[_rows(_BN, L), _full((L, 2 * L))],
    out_specs=[_rows(_BN, L), _rows(_BN, L)],
    out_shape=[jax.ShapeDtypeStruct((N, L), jnp.float32),
               jax.ShapeDtypeStruct((N, L), jnp.float32)],
    compiler_params=_params,
)

_edge_mlp = pl.pallas_call(
    _edge_body,
    grid=(E // _BE,),
    in_specs=[_rows(_BE, L), _rows(_BE, L), _full((L, L)), _full((1, L)),
              _full((L, L)), _full((1, L)), _full((1, L)), _full((1, L))],
    out_specs=[_rows(_BE, L), _rows(_BE, L)],
    out_shape=[jax.ShapeDtypeStruct((E, L), jnp.float32),
               jax.ShapeDtypeStruct((E, L), jnp.float32)],
    compiler_params=_params,
)

_node_mlp = pl.pallas_call(
    _node_body,
    grid=(N // _BN,),
    in_specs=[pl.BlockSpec((NC, _BN, L), lambda j: (0, j, 0)),
              _rows(_BN, L), _full((L, L)), _full((L, L)), _full((1, L)),
              _full((L, L)), _full((1, L)), _full((1, L)), _full((1, L))],
    out_specs=_rows(_BN, L),
    out_shape=jax.ShapeDtypeStruct((N, L), jnp.float32),
    compiler_params=_params,
)

_dec = pl.pallas_call(
    _dec_body,
    grid=(N // _BN,),
    in_specs=[_rows(_BN, L), _full((L, L)), _full((1, L)), _full((L, 3)), _full((1, 3))],
    out_specs=_rows(_BN, 3),
    out_shape=jax.ShapeDtypeStruct((N, 3), jnp.float32),
    compiler_params=_params,
)


def kernel(x, edge_index, edge_attr, enc_node_W, enc_node_b, enc_edge_W, enc_edge_b,
           edge_W1, edge_b1, edge_W2, edge_b2, edge_ln_g, edge_ln_b,
           node_W1, node_b1, node_W2, node_b2, node_ln_g, node_ln_b,
           dec_W1, dec_b1, dec_W2, dec_b2):
    src = edge_index[0].astype(jnp.int32)
    dst = edge_index[1].astype(jnp.int32)

    node_lat = _enc_node(x, enc_node_W, enc_node_b.reshape(1, L))
    edge_lat = _enc_edge(edge_attr, enc_edge_W, enc_edge_b.reshape(1, L))

    zeros_tile = jnp.zeros((CH, L), jnp.float32)

    w1a = edge_W1[:, :L, :]
    wbc = jnp.concatenate([edge_W1[:, L:2 * L, :], edge_W1[:, 2 * L:, :]], axis=2)
    xs = (w1a, wbc, edge_b1.reshape(STEPS, 1, L), edge_W2,
          edge_b2.reshape(STEPS, 1, L), edge_ln_g.reshape(STEPS, 1, L),
          edge_ln_b.reshape(STEPS, 1, L),
          node_W1[:, :L, :], node_W1[:, L:, :], node_b1.reshape(STEPS, 1, L),
          node_W2, node_b2.reshape(STEPS, 1, L), node_ln_g.reshape(STEPS, 1, L),
          node_ln_b.reshape(STEPS, 1, L))

    def step(carry, ws):
        nl, el = carry
        (sw1a, swbc, sb1, sw2, sb2, slg, slb,
         snw1a, snw1b, snb1, snw2, snb2, snlg, snlb) = ws
        a_t, b_t = _pre(nl, swbc)
        gath = _sc_gather_add(a_t, b_t, src, dst)
        e_upd, el_new = _edge_mlp(el, gath, sw1a, sb1, sw2, sb2, slg, slb)
        parts = _sc_segment_sum(e_upd, dst, zeros_tile)
        nl_new = _node_mlp(parts, nl, snw1a, snw1b, snb1, snw2, snb2, snlg, snlb)
        return (nl_new, el_new), None

    (node_lat, _), _ = lax.scan(step, (node_lat, edge_lat), xs)

    return _dec(node_lat, dec_W1, dec_b1.reshape(1, L), dec_W2, dec_b2.reshape(1, 3))


# speedup vs baseline: 1.2647x; 1.0718x over previous
import functools

import jax
import jax.numpy as jnp
from jax import lax
from jax.experimental import pallas as pl
from jax.experimental.pallas import tpu as pltpu
from jax.experimental.pallas import tpu_sc as plsc

N = 10000
E = 160000
L = 128
STEPS = 8

NC = 2
NS = 16
NW = NC * NS
CH = 128
NCH = E // CH
N_PAD = 10240
ROWS_PER_TILE = N_PAD // NS

_sc_mesh = plsc.VectorSubcoreMesh(core_axis_name="c", subcore_axis_name="s")

K = 5
_BIGW = 26
_BIG = 40
_SMALL = 35
IDXW = _BIG * CH


def _worker_span(wid):
    start_c = _BIG * wid - (_BIG - _SMALL) * jnp.maximum(0, wid - _BIGW)
    ngroups = jnp.where(wid < _BIGW, _BIG // K, _SMALL // K)
    return start_c, ngroups


@functools.partial(
    pl.kernel,
    out_type=jax.ShapeDtypeStruct((E, L), jnp.float32),
    mesh=_sc_mesh,
    scratch_types=[
        pltpu.VMEM((K, CH), jnp.int32),
        pltpu.VMEM((K, CH), jnp.int32),
        pltpu.VMEM((K, CH, L), jnp.float32),
        pltpu.SemaphoreType.DMA,
        pltpu.SemaphoreType.DMA,
        pltpu.SemaphoreType.DMA,
    ],
)
def _sc_gather_add(a_hbm, b_hbm, src_hbm, dst_hbm, out_hbm, si_v, di_v, buf_v,
                   sem_i, sem, sem_w):
    wid = lax.axis_index("s") * NC + lax.axis_index("c")
    start_c, ngroups = _worker_span(wid)
    row0 = start_c * CH

    def group(g, _):
        local = g * (K * CH)
        d = []
        for b in range(K):
            r = pl.ds(row0 + local + b * CH, CH)
            d.append(pltpu.async_copy(src_hbm.at[r], si_v.at[b], sem_i))
            d.append(pltpu.async_copy(dst_hbm.at[r], di_v.at[b], sem_i))

        @pl.when(g > 0)
        def _drain_wb():
            for b in range(K):
                pltpu.make_async_copy(
                    buf_v.at[b], out_hbm.at[pl.ds(row0, CH)], sem_w).wait()

        for w in d:
            w.wait()
        d = []
        for b in range(K):
            d.append(pltpu.async_copy(a_hbm.at[si_v.at[b]], buf_v.at[b], sem))
        for w in d:
            w.wait()
        d = []
        for b in range(K):
            d.append(pltpu.async_copy(b_hbm.at[di_v.at[b]], buf_v.at[b], sem, add=True))
        for w in d:
            w.wait()
        for b in range(K):
            pltpu.async_copy(
                buf_v.at[b], out_hbm.at[pl.ds(row0 + local + b * CH, CH)], sem_w)
        return 0

    lax.fori_loop(0, ngroups, group, 0)
    for b in range(K):
        pltpu.make_async_copy(buf_v.at[b], out_hbm.at[pl.ds(row0, CH)], sem_w).wait()


KS = 2


@functools.partial(
    pl.kernel,
    out_type=jax.ShapeDtypeStruct((NC, N_PAD, L), jnp.float32),
    mesh=_sc_mesh,
    scratch_types=[
        pltpu.VMEM((KS, CH), jnp.int32),
        pltpu.VMEM((KS, CH, L), jnp.float32),
        pltpu.VMEM_SHARED((N_PAD, L), jnp.float32),
        pltpu.SemaphoreType.DMA,
        pltpu.SemaphoreType.DMA,
    ],
)
def _sc_segment_sum(vals_hbm, dst_hbm, zeros_hbm, out_hbm, idx_v, buf_v, acc_sh,
                    sem_i, sem_v):
    cid = lax.axis_index("c")
    sid = lax.axis_index("s")
    wid = sid * NC + cid
    start_c = 40 * wid - 2 * jnp.maximum(0, wid - 17)
    ngroups = jnp.where(wid < 17, 40 // KS, 38 // KS)
    row0 = start_c * CH

    pltpu.sync_copy(zeros_hbm, buf_v.at[0])
    for b in range(ROWS_PER_TILE // CH):
        pltpu.sync_copy(buf_v.at[0],
                        acc_sh.at[pl.ds(sid * ROWS_PER_TILE + b * CH, CH)])
    plsc.subcore_barrier()

    def group(g, _):
        local = g * (KS * CH)
        d = []
        for b in range(KS):
            r = pl.ds(row0 + local + b * CH, CH)
            d.append(pltpu.async_copy(dst_hbm.at[r], idx_v.at[b], sem_i))
            d.append(pltpu.async_copy(vals_hbm.at[r], buf_v.at[b], sem_v))
        for w in d:
            w.wait()
        d = []
        for b in range(KS):
            d.append(pltpu.async_copy(buf_v.at[b], acc_sh.at[idx_v.at[b]],
                                      sem_v, add=True))
        for w in d:
            w.wait()
        return 0

    lax.fori_loop(0, ngroups, group, 0)
    plsc.subcore_barrier()
    pltpu.sync_copy(
        acc_sh.at[pl.ds(sid * ROWS_PER_TILE, ROWS_PER_TILE)],
        out_hbm.at[cid, pl.ds(sid * ROWS_PER_TILE, ROWS_PER_TILE)],
    )


def _leaky(h):
    return jnp.where(h >= 0, h, h * jnp.float32(0.01))


def _ln(h, g, b):
    mu = jnp.mean(h, axis=-1, keepdims=True)
    d = h - mu
    var = jnp.mean(d * d, axis=-1, keepdims=True)
    return d * lax.rsqrt(var + jnp.float32(1e-5)) * g + b


_BN = 2000
_BE = 4000


def _enc_node_body(x_ref, w_ref, b_ref, o_ref):
    o_ref[...] = jnp.dot(x_ref[...], w_ref[...], preferred_element_type=jnp.float32) + b_ref[...]


def _pre_body(nl_ref, wcat_ref, a_ref, b_ref):
    acc = jnp.dot(nl_ref[...], wcat_ref[...], preferred_element_type=jnp.float32)
    a_ref[...] = acc[:, :L]
    b_ref[...] = acc[:, L:]


def _edge_tail(e, g_ref, w1_ref, b1_ref, w2_ref, b2_ref, lg_ref, lb_ref):
    h = jnp.dot(e, w1_ref[...], preferred_element_type=jnp.float32) + g_ref[...] + b1_ref[...]
    h = _leaky(h)
    h = jnp.dot(h, w2_ref[...], preferred_element_type=jnp.float32) + b2_ref[...]
    h = _leaky(h)
    return _ln(h, lg_ref[...], lb_ref[...])


def _edge_body(e_ref, g_ref, w1_ref, b1_ref, w2_ref, b2_ref, lg_ref, lb_ref, up_ref, new_ref):
    e = e_ref[...]
    up = _edge_tail(e, g_ref, w1_ref, b1_ref, w2_ref, b2_ref, lg_ref, lb_ref)
    up_ref[...] = up
    new_ref[...] = e + up


def _edge_first_body(at_ref, ew_ref, eb_ref, g_ref, w1_ref, b1_ref, w2_ref, b2_ref,
                     lg_ref, lb_ref, up_ref, new_ref):
    e = (jnp.dot(at_ref[...], ew_ref[...], preferred_element_type=jnp.float32)
         + eb_ref[...])
    up = _edge_tail(e, g_ref, w1_ref, b1_ref, w2_ref, b2_ref, lg_ref, lb_ref)
    up_ref[...] = up
    new_ref[...] = e + up


def _edge_last_body(e_ref, g_ref, w1_ref, b1_ref, w2_ref, b2_ref, lg_ref, lb_ref, up_ref):
    up_ref[...] = _edge_tail(e_ref[...], g_ref, w1_ref, b1_ref, w2_ref, b2_ref,
                             lg_ref, lb_ref)


def _node_body(p_ref, nl_ref, w1a_ref, w1b_ref, b1_ref, w2_ref, b2_ref, lg_ref, lb_ref, o_ref):
    nl = nl_ref[...]
    agg = p_ref[0] + p_ref[1]
    h = (jnp.dot(nl, w1a_ref[...], preferred_element_type=jnp.float32)
         + jnp.dot(agg, w1b_ref[...], preferred_element_type=jnp.float32)
         + b1_ref[...])
    h = _leaky(h)
    h = jnp.dot(h, w2_ref[...], preferred_element_type=jnp.float32) + b2_ref[...]
    h = _leaky(h)
    o_ref[...] = nl + _ln(h, lg_ref[...], lb_ref[...])


def _dec_body(nl_ref, w1_ref, b1_ref, w2_ref, b2_ref, o_ref):
    h = jnp.dot(nl_ref[...], w1_ref[...], preferred_element_type=jnp.float32) + b1_ref[...]
    h = _leaky(h)
    o_ref[...] = jnp.dot(h, w2_ref[...], preferred_element_type=jnp.float32) + b2_ref[...]


def _full(shape):
    return pl.BlockSpec(shape, lambda j: tuple(0 for _ in shape))


def _rows(block, width):
    return pl.BlockSpec((block, width), lambda j: (j, 0))


_params = pltpu.CompilerParams(dimension_semantics=("parallel",))

_enc_node = pl.pallas_call(
    _enc_node_body,
    grid=(N // _BN,),
    in_specs=[_rows(_BN, L), _full((L, L)), _full((1, L))],
    out_specs=_rows(_BN, L),
    out_shape=jax.ShapeDtypeStruct((N, L), jnp.float32),
    compiler_params=_params,
)

_pre = pl.pallas_call(
    _pre_body,
    grid=(N // _BN,),
    in_specs=[_rows(_BN, L), _full((L, 2 * L))],
    out_specs=[_rows(_BN, L), _rows(_BN, L)],
    out_shape=[jax.ShapeDtypeStruct((N, L), jnp.float32),
               jax.ShapeDtypeStruct((N, L), jnp.float32)],
    compiler_params=_params,
)

_edge_mlp = pl.pallas_call(
    _edge_body,
    grid=(E // _BE,),
    in_specs=[_rows(_BE, L), _rows(_BE, L), _full((L, L)), _full((1, L)),
              _full((L, L)), _full((1, L)), _full((1, L)), _full((1, L))],
    out_specs=[_rows(_BE, L), _rows(_BE, L)],
    out_shape=[jax.ShapeDtypeStruct((E, L), jnp.float32),
               jax.ShapeDtypeStruct((E, L), jnp.float32)],
    compiler_params=_params,
)

_edge_mlp_first = pl.pallas_call(
    _edge_first_body,
    grid=(E // _BE,),
    in_specs=[_rows(_BE, 4), _full((4, L)), _full((1, L)),
              _rows(_BE, L), _full((L, L)), _full((1, L)),
              _full((L, L)), _full((1, L)), _full((1, L)), _full((1, L))],
    out_specs=[_rows(_BE, L), _rows(_BE, L)],
    out_shape=[jax.ShapeDtypeStruct((E, L), jnp.float32),
               jax.ShapeDtypeStruct((E, L), jnp.float32)],
    compiler_params=_params,
)

_edge_mlp_last = pl.pallas_call(
    _edge_last_body,
    grid=(E // _BE,),
    in_specs=[_rows(_BE, L), _rows(_BE, L), _full((L, L)), _full((1, L)),
              _full((L, L)), _full((1, L)), _full((1, L)), _full((1, L))],
    out_specs=_rows(_BE, L),
    out_shape=jax.ShapeDtypeStruct((E, L), jnp.float32),
    compiler_params=_params,
)

_node_mlp = pl.pallas_call(
    _node_body,
    grid=(N // _BN,),
    in_specs=[pl.BlockSpec((NC, _BN, L), lambda j: (0, j, 0)),
              _rows(_BN, L), _full((L, L)), _full((L, L)), _full((1, L)),
              _full((L, L)), _full((1, L)), _full((1, L)), _full((1, L))],
    out_specs=_rows(_BN, L),
    out_shape=jax.ShapeDtypeStruct((N, L), jnp.float32),
    compiler_params=_params,
)

_dec = pl.pallas_call(
    _dec_body,
    grid=(N // _BN,),
    in_specs=[_rows(_BN, L), _full((L, L)), _full((1, L)), _full((L, 3)), _full((1, 3))],
    out_specs=_rows(_BN, 3),
    out_shape=jax.ShapeDtypeStruct((N, 3), jnp.float32),
    compiler_params=_params,
)


def kernel(x, edge_index, edge_attr, enc_node_W, enc_node_b, enc_edge_W, enc_edge_b,
           edge_W1, edge_b1, edge_W2, edge_b2, edge_ln_g, edge_ln_b,
           node_W1, node_b1, node_W2, node_b2, node_ln_g, node_ln_b,
           dec_W1, dec_b1, dec_W2, dec_b2):
    src = edge_index[0].astype(jnp.int32)
    dst = edge_index[1].astype(jnp.int32)

    node_lat = _enc_node(x, enc_node_W, enc_node_b.reshape(1, L))

    zeros_tile = jnp.zeros((CH, L), jnp.float32)

    w1a = edge_W1[:, :L, :]
    wbc = jnp.concatenate([edge_W1[:, L:2 * L, :], edge_W1[:, 2 * L:, :]], axis=2)
    xs = (w1a, wbc, edge_b1.reshape(STEPS, 1, L), edge_W2,
          edge_b2.reshape(STEPS, 1, L), edge_ln_g.reshape(STEPS, 1, L),
          edge_ln_b.reshape(STEPS, 1, L),
          node_W1[:, :L, :], node_W1[:, L:, :], node_b1.reshape(STEPS, 1, L),
          node_W2, node_b2.reshape(STEPS, 1, L), node_ln_g.reshape(STEPS, 1, L),
          node_ln_b.reshape(STEPS, 1, L))

    def _sparse_step(nl, ws, edge_call):
        (sw1a, swbc, sb1, sw2, sb2, slg, slb,
         snw1a, snw1b, snb1, snw2, snb2, snlg, snlb) = ws
        a_t, b_t = _pre(nl, swbc)
        gath = _sc_gather_add(a_t, b_t, src, dst)
        e_upd, el_new = edge_call(gath, sw1a, sb1, sw2, sb2, slg, slb)
        parts = _sc_segment_sum(e_upd, dst, zeros_tile)
        nl_new = _node_mlp(parts, nl, snw1a, snw1b, snb1, snw2, snb2, snlg, snlb)
        return nl_new, el_new

    ws0 = jax.tree.map(lambda t: t[0], xs)
    eb = enc_edge_b.reshape(1, L)
    node_lat, edge_lat = _sparse_step(
        node_lat, ws0,
        lambda g, *w: _edge_mlp_first(edge_attr, enc_edge_W, eb, g, *w))

    mid = jax.tree.map(lambda t: t[1:STEPS - 1], xs)

    def step(carry, ws):
        nl, el = carry
        nl2, el2 = _sparse_step(nl, ws, lambda g, *w: _edge_mlp(el, g, *w))
        return (nl2, el2), None

    (node_lat, edge_lat), _ = lax.scan(step, (node_lat, edge_lat), mid)

    ws7 = jax.tree.map(lambda t: t[STEPS - 1], xs)
    node_lat, _ = _sparse_step(
        node_lat, ws7,
        lambda g, *w: (_edge_mlp_last(edge_lat, g, *w), None))

    return _dec(node_lat, dec_W1, dec_b1.reshape(1, L), dec_W2, dec_b2.reshape(1, 3))
